# Initial kernel scaffold; baseline (speedup 1.0000x reference)
import jax
import jax.numpy as jnp
from jax.experimental import pallas as pl


def kernel(x, edge_index, W1, b1, W2, b2, prelu_a):
    raise NotImplementedError("write your pallas kernel here")


# speedup vs baseline: 17.9437x; 17.9437x over previous
import functools

import jax
import jax.numpy as jnp
from jax import lax
from jax.experimental import pallas as pl
from jax.experimental.pallas import tpu as pltpu
from jax.experimental.pallas import tpu_sc as plsc

N_NODES = 10000
D = 128
N_EDGES = 320000

NC, NS = 2, 16
NW = NC * NS
EPT = N_EDGES // NW
CHUNK = 128
NFULL = EPT // CHUNK
REM = EPT - NFULL * CHUNK
ROWS_BLK = 1024
NBLK = 10
N_PAD = ROWS_BLK * NBLK
ROWS_PER_TILE = N_PAD // NS
ZROWS = 128

_mesh = plsc.VectorSubcoreMesh(core_axis_name="c", subcore_axis_name="s")


def _sc_deg_body(dst_hbm, degp_hbm, dst_v, deg_v):
    c = lax.axis_index("c")
    s = lax.axis_index("s")
    wid = c * NS + s
    zero16 = jnp.zeros((16,), jnp.float32)
    one16 = jnp.ones((16,), jnp.float32)

    def zero_body(i, _):
        deg_v[pl.ds(i * 16, 16)] = zero16
        return 0

    lax.fori_loop(0, N_PAD // 16, zero_body, 0)
    pltpu.sync_copy(dst_hbm.at[pl.ds(wid * EPT, EPT)], dst_v)

    def scat_body(i, _):
        idx = dst_v[pl.ds(i * 16, 16)]
        plsc.addupdate_scatter(deg_v, [idx], one16)
        return 0

    lax.fori_loop(0, EPT // 16, scat_body, 0)
    for j in range(NBLK):
        pltpu.sync_copy(deg_v.at[pl.ds(j * ROWS_BLK, ROWS_BLK)],
                        degp_hbm.at[j, wid])


@functools.partial(
    pl.kernel,
    out_type=jax.ShapeDtypeStruct((NBLK, NW, ROWS_BLK), jnp.float32),
    mesh=_mesh,
    scratch_types=[
        pltpu.VMEM((EPT,), jnp.int32),
        pltpu.VMEM((N_PAD,), jnp.float32),
    ],
    compiler_params=pltpu.CompilerParams(needs_layout_passes=False),
)
def _sc_deg(dst_hbm, degp_hbm, dst_v, deg_v):
    _sc_deg_body(dst_hbm, degp_hbm, dst_v, deg_v)


def _sc_scatter_body(y_hbm, src_hbm, dst_hbm, accp_hbm,
                     src_v, dst_v, src16_v, dst16_v, rows_v, rows16_v,
                     zero_v, acc_sh, gsem):
    c = lax.axis_index("c")
    s = lax.axis_index("s")
    zero16 = jnp.zeros((16,), jnp.float32)

    def zfill(i, _):
        r = i // (D // 16)
        col = (i % (D // 16)) * 16
        zero_v[r, pl.ds(col, 16)] = zero16
        return 0

    lax.fori_loop(0, ZROWS * (D // 16), zfill, 0)
    for k in range(ROWS_PER_TILE // ZROWS):
        pltpu.sync_copy(zero_v, acc_sh.at[pl.ds(s * ROWS_PER_TILE + k * ZROWS, ZROWS)])
    plsc.subcore_barrier()

    base = (c * NS + s) * EPT

    def body(i, _):
        off = base + i * CHUNK
        pltpu.sync_copy(src_hbm.at[pl.ds(off, CHUNK)], src_v)
        pltpu.async_copy(y_hbm.at[src_v], rows_v, gsem).wait()
        pltpu.sync_copy(dst_hbm.at[pl.ds(off, CHUNK)], dst_v)
        pltpu.sync_copy(rows_v, acc_sh.at[dst_v], add=True)
        return 0

    lax.fori_loop(0, NFULL, body, 0)

    off = base + NFULL * CHUNK
    pltpu.sync_copy(src_hbm.at[pl.ds(off, REM)], src16_v)
    pltpu.async_copy(y_hbm.at[src16_v], rows16_v, gsem).wait()
    pltpu.sync_copy(dst_hbm.at[pl.ds(off, REM)], dst16_v)
    pltpu.sync_copy(rows16_v, acc_sh.at[dst16_v], add=True)

    plsc.subcore_barrier()
    pltpu.sync_copy(acc_sh.at[pl.ds(s * ROWS_PER_TILE, ROWS_PER_TILE)],
                    accp_hbm.at[c, pl.ds(s * ROWS_PER_TILE, ROWS_PER_TILE)])


@functools.partial(
    pl.kernel,
    out_type=jax.ShapeDtypeStruct((NC, N_PAD, D), jnp.float32),
    mesh=_mesh,
    scratch_types=[
        pltpu.VMEM((CHUNK,), jnp.int32),
        pltpu.VMEM((CHUNK,), jnp.int32),
        pltpu.VMEM((REM,), jnp.int32),
        pltpu.VMEM((REM,), jnp.int32),
        pltpu.VMEM((CHUNK, D), jnp.float32),
        pltpu.VMEM((REM, D), jnp.float32),
        pltpu.VMEM((ZROWS, D), jnp.float32),
        pltpu.VMEM_SHARED((N_PAD, D), jnp.float32),
        pltpu.SemaphoreType.DMA,
    ],
    compiler_params=pltpu.CompilerParams(needs_layout_passes=False),
)
def _sc_scatter(y_hbm, src_hbm, dst_hbm, accp_hbm,
                src_v, dst_v, src16_v, dst16_v, rows_v, rows16_v,
                zero_v, acc_sh, gsem):
    _sc_scatter_body(y_hbm, src_hbm, dst_hbm, accp_hbm,
                     src_v, dst_v, src16_v, dst16_v, rows_v, rows16_v,
                     zero_v, acc_sh, gsem)


def _tc_a_body(x_ref, w_ref, degp_ref, xw_ref, y_ref, db_ref):
    deg = 1.0 + jnp.sum(degp_ref[0], axis=0, keepdims=True)
    dinv = lax.rsqrt(deg)
    db = lax.dot_general(dinv, jnp.ones((1, D), jnp.float32),
                         (((0,), (0,)), ((), ())),
                         preferred_element_type=jnp.float32)
    xw = jnp.dot(x_ref[...], w_ref[...], preferred_element_type=jnp.float32)
    xw_ref[...] = xw
    db_ref[...] = db
    y_ref[...] = db * xw


def _tc_a(x, W1, degp):
    return pl.pallas_call(
        _tc_a_body,
        grid=(NBLK,),
        in_specs=[
            pl.BlockSpec((ROWS_BLK, D), lambda i: (i, 0)),
            pl.BlockSpec((D, D), lambda i: (0, 0)),
            pl.BlockSpec((1, NW, ROWS_BLK), lambda i: (i, 0, 0)),
        ],
        out_specs=[pl.BlockSpec((ROWS_BLK, D), lambda i: (i, 0))] * 3,
        out_shape=[jax.ShapeDtypeStruct((N_NODES, D), jnp.float32)] * 3,
    )(x, W1, degp)


def _tc_mid_body(accp_ref, db_ref, xw1_ref, b_ref, a_ref, w_ref,
                 xw2_ref, y2_ref):
    acc = accp_ref[0] + accp_ref[1]
    db = db_ref[...]
    z = db * acc + db * db * xw1_ref[...] + b_ref[...]
    h = jnp.where(z >= 0, z, a_ref[...] * z)
    xw2 = jnp.dot(h, w_ref[...], preferred_element_type=jnp.float32)
    xw2_ref[...] = xw2
    y2_ref[...] = db * xw2


def _tc_mid(accp, db, xw1, b1, a_b, W2):
    return pl.pallas_call(
        _tc_mid_body,
        grid=(NBLK,),
        in_specs=[
            pl.BlockSpec((NC, ROWS_BLK, D), lambda i: (0, i, 0)),
            pl.BlockSpec((ROWS_BLK, D), lambda i: (i, 0)),
            pl.BlockSpec((ROWS_BLK, D), lambda i: (i, 0)),
            pl.BlockSpec((1, D), lambda i: (0, 0)),
            pl.BlockSpec((1, D), lambda i: (0, 0)),
            pl.BlockSpec((D, D), lambda i: (0, 0)),
        ],
        out_specs=[pl.BlockSpec((ROWS_BLK, D), lambda i: (i, 0))] * 2,
        out_shape=[jax.ShapeDtypeStruct((N_NODES, D), jnp.float32)] * 2,
    )(accp, db, xw1, b1, a_b, W2)


def _tc_out_body(accp_ref, db_ref, xw2_ref, b_ref, a_ref, out_ref):
    acc = accp_ref[0] + accp_ref[1]
    db = db_ref[...]
    z = db * acc + db * db * xw2_ref[...] + b_ref[...]
    out_ref[...] = jnp.where(z >= 0, z, a_ref[...] * z)


def _tc_out(accp, db, xw2, b2, a_b):
    return pl.pallas_call(
        _tc_out_body,
        grid=(NBLK,),
        in_specs=[
            pl.BlockSpec((NC, ROWS_BLK, D), lambda i: (0, i, 0)),
            pl.BlockSpec((ROWS_BLK, D), lambda i: (i, 0)),
            pl.BlockSpec((ROWS_BLK, D), lambda i: (i, 0)),
            pl.BlockSpec((1, D), lambda i: (0, 0)),
            pl.BlockSpec((1, D), lambda i: (0, 0)),
        ],
        out_specs=pl.BlockSpec((ROWS_BLK, D), lambda i: (i, 0)),
        out_shape=jax.ShapeDtypeStruct((N_NODES, D), jnp.float32),
    )(accp, db, xw2, b2, a_b)


def kernel(x, edge_index, W1, b1, W2, b2, prelu_a):
    src = edge_index[0]
    dst = edge_index[1]
    b1r = jnp.reshape(b1, (1, D))
    b2r = jnp.reshape(b2, (1, D))
    a_b = jnp.broadcast_to(jnp.reshape(prelu_a, (1, 1)), (1, D))

    degp = _sc_deg(dst)
    xw1, y1, db = _tc_a(x, W1, degp)
    accp1 = _sc_scatter(y1, src, dst)
    xw2, y2 = _tc_mid(accp1, db, xw1, b1r, a_b, W2)
    accp2 = _sc_scatter(y2, src, dst)
    return _tc_out(accp2, db, xw2, b2r, a_b)


# speedup vs baseline: 27.3619x; 1.5249x over previous
import functools

import jax
import jax.numpy as jnp
from jax import lax
from jax.experimental import pallas as pl
from jax.experimental.pallas import tpu as pltpu
from jax.experimental.pallas import tpu_sc as plsc

N_NODES = 10000
D = 128
N_EDGES = 320000

NC, NS = 2, 16
NW = NC * NS
EPT = N_EDGES // NW
CHUNK = 128
NFULL = EPT // CHUNK
REM = EPT - NFULL * CHUNK
ROWS_BLK = 1024
NBLK = 10
N_PAD = ROWS_BLK * NBLK
ROWS_PER_TILE = N_PAD // NS
ZROWS = 128

_mesh = plsc.VectorSubcoreMesh(core_axis_name="c", subcore_axis_name="s")


def _sc_deg_body(dst_hbm, degp_hbm, dst_v, deg_v):
    c = lax.axis_index("c")
    s = lax.axis_index("s")
    wid = c * NS + s
    zero16 = jnp.zeros((16,), jnp.float32)
    one16 = jnp.ones((16,), jnp.float32)

    def zero_body(i, _):
        deg_v[pl.ds(i * 16, 16)] = zero16
        return 0

    lax.fori_loop(0, N_PAD // 16, zero_body, 0)
    pltpu.sync_copy(dst_hbm.at[pl.ds(wid * EPT, EPT)], dst_v)

    def scat_body(i, _):
        idx = dst_v[pl.ds(i * 16, 16)]
        plsc.addupdate_scatter(deg_v, [idx], one16)
        return 0

    lax.fori_loop(0, EPT // 16, scat_body, 0)
    for j in range(NBLK):
        pltpu.sync_copy(deg_v.at[pl.ds(j * ROWS_BLK, ROWS_BLK)],
                        degp_hbm.at[j, wid])


@functools.partial(
    pl.kernel,
    out_type=jax.ShapeDtypeStruct((NBLK, NW, ROWS_BLK), jnp.float32),
    mesh=_mesh,
    scratch_types=[
        pltpu.VMEM((EPT,), jnp.int32),
        pltpu.VMEM((N_PAD,), jnp.float32),
    ],
    compiler_params=pltpu.CompilerParams(needs_layout_passes=False),
)
def _sc_deg(dst_hbm, degp_hbm, dst_v, deg_v):
    _sc_deg_body(dst_hbm, degp_hbm, dst_v, deg_v)


def _sc_scatter_body(y_hbm, src_hbm, dst_hbm, accp_hbm,
                     src_v0, src_v1, dst_v0, dst_v1, dst16_v,
                     rows0, rows1, acc_sh, gsem):
    c = lax.axis_index("c")
    s = lax.axis_index("s")
    wid = c * NS + s
    zero16 = jnp.zeros((16,), jnp.float32)

    def zfill(i, _):
        r = i // (D // 16)
        col = (i % (D // 16)) * 16
        rows0[r, pl.ds(col, 16)] = zero16
        return 0

    lax.fori_loop(0, ZROWS * (D // 16), zfill, 0)
    for k in range(ROWS_PER_TILE // ZROWS):
        pltpu.sync_copy(rows0, acc_sh.at[pl.ds(s * ROWS_PER_TILE + k * ZROWS, ZROWS)])
    plsc.subcore_barrier()

    base = wid * EPT

    pltpu.sync_copy(src_hbm.at[pl.ds(base, CHUNK)], src_v0)
    pltpu.async_copy(y_hbm.at[src_v0], rows0, gsem)

    def body(j, _):
        i0 = 2 * j
        off = base + i0 * CHUNK
        pltpu.sync_copy(src_hbm.at[pl.ds(off + CHUNK, CHUNK)], src_v1)
        pltpu.make_async_copy(y_hbm.at[src_v0], rows0, gsem).wait()
        pltpu.async_copy(y_hbm.at[src_v1], rows1, gsem)
        pltpu.sync_copy(dst_hbm.at[pl.ds(off, CHUNK)], dst_v0)
        pltpu.sync_copy(rows0, acc_sh.at[dst_v0], add=True)

        @pl.when(i0 + 2 < NFULL)
        def _():
            pltpu.sync_copy(src_hbm.at[pl.ds(off + 2 * CHUNK, CHUNK)], src_v0)

        pltpu.make_async_copy(y_hbm.at[src_v1], rows1, gsem).wait()

        @pl.when(i0 + 2 < NFULL)
        def _():
            pltpu.async_copy(y_hbm.at[src_v0], rows0, gsem)

        pltpu.sync_copy(dst_hbm.at[pl.ds(off + CHUNK, CHUNK)], dst_v1)
        pltpu.sync_copy(rows1, acc_sh.at[dst_v1], add=True)
        return 0

    lax.fori_loop(0, NFULL // 2, body, 0)

    off = base + NFULL * CHUNK
    pltpu.sync_copy(src_hbm.at[pl.ds(off, REM)], src_v0.at[pl.ds(0, REM)])
    pltpu.async_copy(y_hbm.at[src_v0.at[pl.ds(0, REM)]],
                     rows0.at[pl.ds(0, REM)], gsem).wait()
    pltpu.sync_copy(dst_hbm.at[pl.ds(off, REM)], dst16_v)
    pltpu.sync_copy(rows0.at[pl.ds(0, REM)], acc_sh.at[dst16_v], add=True)

    plsc.subcore_barrier()
    pltpu.sync_copy(acc_sh.at[pl.ds(s * ROWS_PER_TILE, ROWS_PER_TILE)],
                    accp_hbm.at[c, pl.ds(s * ROWS_PER_TILE, ROWS_PER_TILE)])


@functools.partial(
    pl.kernel,
    out_type=jax.ShapeDtypeStruct((NC, N_PAD, D), jnp.float32),
    mesh=_mesh,
    scratch_types=[
        pltpu.VMEM((CHUNK,), jnp.int32),
        pltpu.VMEM((CHUNK,), jnp.int32),
        pltpu.VMEM((CHUNK,), jnp.int32),
        pltpu.VMEM((CHUNK,), jnp.int32),
        pltpu.VMEM((REM,), jnp.int32),
        pltpu.VMEM((CHUNK, D), jnp.float32),
        pltpu.VMEM((CHUNK, D), jnp.float32),
        pltpu.VMEM_SHARED((N_PAD, D), jnp.float32),
        pltpu.SemaphoreType.DMA,
    ],
    compiler_params=pltpu.CompilerParams(needs_layout_passes=False),
)
def _sc_scatter(y_hbm, src_hbm, dst_hbm, accp_hbm,
                src_v0, src_v1, dst_v0, dst_v1, dst16_v,
                rows0, rows1, acc_sh, gsem):
    _sc_scatter_body(y_hbm, src_hbm, dst_hbm, accp_hbm,
                     src_v0, src_v1, dst_v0, dst_v1, dst16_v,
                     rows0, rows1, acc_sh, gsem)


def _tc_a_body(x_ref, w_ref, degp_ref, xw_ref, y_ref, db_ref):
    deg = 1.0 + jnp.sum(degp_ref[0], axis=0, keepdims=True)
    dinv = lax.rsqrt(deg)
    db = lax.dot_general(dinv, jnp.ones((1, D), jnp.float32),
                         (((0,), (0,)), ((), ())),
                         preferred_element_type=jnp.float32)
    xw = jnp.dot(x_ref[...], w_ref[...], preferred_element_type=jnp.float32)
    xw_ref[...] = xw
    db_ref[...] = db
    y_ref[...] = db * xw


def _tc_a(x, W1, degp):
    return pl.pallas_call(
        _tc_a_body,
        grid=(NBLK,),
        in_specs=[
            pl.BlockSpec((ROWS_BLK, D), lambda i: (i, 0)),
            pl.BlockSpec((D, D), lambda i: (0, 0)),
            pl.BlockSpec((1, NW, ROWS_BLK), lambda i: (i, 0, 0)),
        ],
        out_specs=[pl.BlockSpec((ROWS_BLK, D), lambda i: (i, 0))] * 3,
        out_shape=[jax.ShapeDtypeStruct((N_NODES, D), jnp.float32)] * 3,
    )(x, W1, degp)


def _tc_mid_body(accp_ref, db_ref, xw1_ref, b_ref, a_ref, w_ref,
                 xw2_ref, y2_ref):
    acc = accp_ref[0] + accp_ref[1]
    db = db_ref[...]
    z = db * acc + db * db * xw1_ref[...] + b_ref[...]
    h = jnp.where(z >= 0, z, a_ref[...] * z)
    xw2 = jnp.dot(h, w_ref[...], preferred_element_type=jnp.float32)
    xw2_ref[...] = xw2
    y2_ref[...] = db * xw2


def _tc_mid(accp, db, xw1, b1, a_b, W2):
    return pl.pallas_call(
        _tc_mid_body,
        grid=(NBLK,),
        in_specs=[
            pl.BlockSpec((NC, ROWS_BLK, D), lambda i: (0, i, 0)),
            pl.BlockSpec((ROWS_BLK, D), lambda i: (i, 0)),
            pl.BlockSpec((ROWS_BLK, D), lambda i: (i, 0)),
            pl.BlockSpec((1, D), lambda i: (0, 0)),
            pl.BlockSpec((1, D), lambda i: (0, 0)),
            pl.BlockSpec((D, D), lambda i: (0, 0)),
        ],
        out_specs=[pl.BlockSpec((ROWS_BLK, D), lambda i: (i, 0))] * 2,
        out_shape=[jax.ShapeDtypeStruct((N_NODES, D), jnp.float32)] * 2,
    )(accp, db, xw1, b1, a_b, W2)


def _tc_out_body(accp_ref, db_ref, xw2_ref, b_ref, a_ref, out_ref):
    acc = accp_ref[0] + accp_ref[1]
    db = db_ref[...]
    z = db * acc + db * db * xw2_ref[...] + b_ref[...]
    out_ref[...] = jnp.where(z >= 0, z, a_ref[...] * z)


def _tc_out(accp, db, xw2, b2, a_b):
    return pl.pallas_call(
        _tc_out_body,
        grid=(NBLK,),
        in_specs=[
            pl.BlockSpec((NC, ROWS_BLK, D), lambda i: (0, i, 0)),
            pl.BlockSpec((ROWS_BLK, D), lambda i: (i, 0)),
            pl.BlockSpec((ROWS_BLK, D), lambda i: (i, 0)),
            pl.BlockSpec((1, D), lambda i: (0, 0)),
            pl.BlockSpec((1, D), lambda i: (0, 0)),
        ],
        out_specs=pl.BlockSpec((ROWS_BLK, D), lambda i: (i, 0)),
        out_shape=jax.ShapeDtypeStruct((N_NODES, D), jnp.float32),
    )(accp, db, xw2, b2, a_b)


def kernel(x, edge_index, W1, b1, W2, b2, prelu_a):
    src = edge_index[0]
    dst = edge_index[1]
    b1r = jnp.reshape(b1, (1, D))
    b2r = jnp.reshape(b2, (1, D))
    a_b = jnp.broadcast_to(jnp.reshape(prelu_a, (1, 1)), (1, D))

    degp = _sc_deg(dst)
    xw1, y1, db = _tc_a(x, W1, degp)
    accp1 = _sc_scatter(y1, src, dst)
    xw2, y2 = _tc_mid(accp1, db, xw1, b1r, a_b, W2)
    accp2 = _sc_scatter(y2, src, dst)
    return _tc_out(accp2, db, xw2, b2r, a_b)


# speedup vs baseline: 30.1756x; 1.1028x over previous
import functools

import jax
import jax.numpy as jnp
from jax import lax
from jax.experimental import pallas as pl
from jax.experimental.pallas import tpu as pltpu
from jax.experimental.pallas import tpu_sc as plsc

N_NODES = 10000
D = 128
N_EDGES = 320000

NC, NS = 2, 16
NW = NC * NS
EPT = N_EDGES // NW
CHUNK = 128
NFULL = EPT // CHUNK
REM = EPT - NFULL * CHUNK
ROWS_BLK = 1024
NBLK = 10
N_PAD = ROWS_BLK * NBLK
ROWS_PER_TILE = N_PAD // NS
ZROWS = 128

_mesh = plsc.VectorSubcoreMesh(core_axis_name="c", subcore_axis_name="s")


def _sc_deg_body(dst_hbm, degp_hbm, dst_v, deg_v):
    c = lax.axis_index("c")
    s = lax.axis_index("s")
    wid = c * NS + s
    zero16 = jnp.zeros((16,), jnp.float32)
    one16 = jnp.ones((16,), jnp.float32)

    def zero_body(i, _):
        deg_v[pl.ds(i * 16, 16)] = zero16
        return 0

    lax.fori_loop(0, N_PAD // 16, zero_body, 0)
    pltpu.sync_copy(dst_hbm.at[pl.ds(wid * EPT, EPT)], dst_v)

    def scat_body(i, _):
        idx = dst_v[pl.ds(i * 16, 16)]
        plsc.addupdate_scatter(deg_v, [idx], one16)
        return 0

    lax.fori_loop(0, EPT // 16, scat_body, 0)
    for j in range(NBLK):
        pltpu.sync_copy(deg_v.at[pl.ds(j * ROWS_BLK, ROWS_BLK)],
                        degp_hbm.at[j, wid])


@functools.partial(
    pl.kernel,
    out_type=jax.ShapeDtypeStruct((NBLK, NW, ROWS_BLK), jnp.float32),
    mesh=_mesh,
    scratch_types=[
        pltpu.VMEM((EPT,), jnp.int32),
        pltpu.VMEM((N_PAD,), jnp.float32),
    ],
    compiler_params=pltpu.CompilerParams(needs_layout_passes=False),
)
def _sc_deg(dst_hbm, degp_hbm, dst_v, deg_v):
    _sc_deg_body(dst_hbm, degp_hbm, dst_v, deg_v)


def _sc_scatter_body(y_hbm, src_hbm, dst_hbm, accp_hbm,
                     src_v0, src_v1, dst_v0, dst_v1, dst_v2, dst16_v,
                     rows0, rows1, acc_sh, gsem, ssem, isem):
    c = lax.axis_index("c")
    s = lax.axis_index("s")
    wid = c * NS + s
    zero16 = jnp.zeros((16,), jnp.float32)

    def zfill(i, _):
        r = i // (D // 16)
        col = (i % (D // 16)) * 16
        rows0[r, pl.ds(col, 16)] = zero16
        return 0

    lax.fori_loop(0, ZROWS * (D // 16), zfill, 0)
    for k in range(ROWS_PER_TILE // ZROWS):
        pltpu.sync_copy(rows0, acc_sh.at[pl.ds(s * ROWS_PER_TILE + k * ZROWS, ZROWS)])
    plsc.subcore_barrier()

    base = wid * EPT
    src_ring = (src_v0, src_v1)
    dst_ring = (dst_v0, dst_v1, dst_v2)
    rows_ring = (rows0, rows1)
    UN = 6
    NJ = NFULL // UN

    pltpu.sync_copy(src_hbm.at[pl.ds(base, CHUNK)], src_v0)
    pltpu.sync_copy(dst_hbm.at[pl.ds(base, CHUNK)], dst_v0)
    pltpu.async_copy(y_hbm.at[src_v0], rows0, gsem)
    pltpu.async_copy(src_hbm.at[pl.ds(base + CHUNK, CHUNK)], src_v1, isem)
    pltpu.async_copy(dst_hbm.at[pl.ds(base + CHUNK, CHUNK)], dst_v1, isem)

    def body(j, _):
        for k in range(UN):
            sv, svn = src_ring[k % 2], src_ring[(k + 1) % 2]
            dv = dst_ring[k % 3]
            rv, rvn = rows_ring[k % 2], rows_ring[(k + 1) % 2]
            svp, dvp = src_ring[(k + 1) % 2], dst_ring[(k + 1) % 3]
            off = base + (j * UN + k) * CHUNK
            last = j * UN + k + 1 >= NFULL

            if k == 5:
                @pl.when(j < NJ - 1)
                def _():
                    pltpu.make_async_copy(src_hbm.at[pl.ds(off + CHUNK, CHUNK)], svp, isem).wait()
                    pltpu.make_async_copy(dst_hbm.at[pl.ds(off + CHUNK, CHUNK)], dvp, isem).wait()
            else:
                pltpu.make_async_copy(src_hbm.at[pl.ds(off + CHUNK, CHUNK)], svp, isem).wait()
                pltpu.make_async_copy(dst_hbm.at[pl.ds(off + CHUNK, CHUNK)], dvp, isem).wait()

            pltpu.make_async_copy(y_hbm.at[sv], rv, gsem).wait()

            pv, pd = rows_ring[(k + 1) % 2], dst_ring[(k + 2) % 3]
            if k == 0:
                @pl.when(j > 0)
                def _():
                    pltpu.make_async_copy(pv, acc_sh.at[pd], ssem).wait()
            else:
                pltpu.make_async_copy(pv, acc_sh.at[pd], ssem).wait()

            if k == 5:
                @pl.when(j < NJ - 1)
                def _():
                    pltpu.async_copy(y_hbm.at[svn], rvn, gsem)
            else:
                pltpu.async_copy(y_hbm.at[svn], rvn, gsem)

            pltpu.async_copy(rv, acc_sh.at[dv], ssem, add=True)

            if k >= 4:
                @pl.when(j < NJ - 1)
                def _():
                    pltpu.async_copy(src_hbm.at[pl.ds(off + 2 * CHUNK, CHUNK)], sv, isem)
                    pltpu.async_copy(dst_hbm.at[pl.ds(off + 2 * CHUNK, CHUNK)], dst_ring[(k + 2) % 3], isem)
            else:
                pltpu.async_copy(src_hbm.at[pl.ds(off + 2 * CHUNK, CHUNK)], sv, isem)
                pltpu.async_copy(dst_hbm.at[pl.ds(off + 2 * CHUNK, CHUNK)], dst_ring[(k + 2) % 3], isem)
        return 0

    lax.fori_loop(0, NJ, body, 0)

    pltpu.make_async_copy(rows1, acc_sh.at[dst_v2], ssem).wait()

    off = base + NFULL * CHUNK
    pltpu.sync_copy(src_hbm.at[pl.ds(off, REM)], src_v0.at[pl.ds(0, REM)])
    pltpu.async_copy(y_hbm.at[src_v0.at[pl.ds(0, REM)]],
                     rows0.at[pl.ds(0, REM)], gsem).wait()
    pltpu.sync_copy(dst_hbm.at[pl.ds(off, REM)], dst16_v)
    pltpu.sync_copy(rows0.at[pl.ds(0, REM)], acc_sh.at[dst16_v], add=True)

    plsc.subcore_barrier()
    pltpu.sync_copy(acc_sh.at[pl.ds(s * ROWS_PER_TILE, ROWS_PER_TILE)],
                    accp_hbm.at[c, pl.ds(s * ROWS_PER_TILE, ROWS_PER_TILE)])


@functools.partial(
    pl.kernel,
    out_type=jax.ShapeDtypeStruct((NC, N_PAD, D), jnp.float32),
    mesh=_mesh,
    scratch_types=[
        pltpu.VMEM((CHUNK,), jnp.int32),
        pltpu.VMEM((CHUNK,), jnp.int32),
        pltpu.VMEM((CHUNK,), jnp.int32),
        pltpu.VMEM((CHUNK,), jnp.int32),
        pltpu.VMEM((CHUNK,), jnp.int32),
        pltpu.VMEM((REM,), jnp.int32),
        pltpu.VMEM((CHUNK, D), jnp.float32),
        pltpu.VMEM((CHUNK, D), jnp.float32),
        pltpu.VMEM_SHARED((N_PAD, D), jnp.float32),
        pltpu.SemaphoreType.DMA,
        pltpu.SemaphoreType.DMA,
        pltpu.SemaphoreType.DMA,
    ],
    compiler_params=pltpu.CompilerParams(needs_layout_passes=False),
)
def _sc_scatter(y_hbm, src_hbm, dst_hbm, accp_hbm,
                src_v0, src_v1, dst_v0, dst_v1, dst_v2, dst16_v,
                rows0, rows1, acc_sh, gsem, ssem, isem):
    _sc_scatter_body(y_hbm, src_hbm, dst_hbm, accp_hbm,
                     src_v0, src_v1, dst_v0, dst_v1, dst_v2, dst16_v,
                     rows0, rows1, acc_sh, gsem, ssem, isem)


def _tc_a_body(x_ref, w_ref, degp_ref, xw_ref, y_ref, db_ref):
    deg = 1.0 + jnp.sum(degp_ref[0], axis=0, keepdims=True)
    dinv = lax.rsqrt(deg)
    db = lax.dot_general(dinv, jnp.ones((1, D), jnp.float32),
                         (((0,), (0,)), ((), ())),
                         preferred_element_type=jnp.float32)
    xw = jnp.dot(x_ref[...], w_ref[...], preferred_element_type=jnp.float32)
    xw_ref[...] = xw
    db_ref[...] = db
    y_ref[...] = db * xw


def _tc_a(x, W1, degp):
    return pl.pallas_call(
        _tc_a_body,
        grid=(NBLK,),
        in_specs=[
            pl.BlockSpec((ROWS_BLK, D), lambda i: (i, 0)),
            pl.BlockSpec((D, D), lambda i: (0, 0)),
            pl.BlockSpec((1, NW, ROWS_BLK), lambda i: (i, 0, 0)),
        ],
        out_specs=[pl.BlockSpec((ROWS_BLK, D), lambda i: (i, 0))] * 3,
        out_shape=[jax.ShapeDtypeStruct((N_NODES, D), jnp.float32)] * 3,
    )(x, W1, degp)


def _tc_mid_body(accp_ref, db_ref, xw1_ref, b_ref, a_ref, w_ref,
                 xw2_ref, y2_ref):
    acc = accp_ref[0] + accp_ref[1]
    db = db_ref[...]
    z = db * acc + db * db * xw1_ref[...] + b_ref[...]
    h = jnp.where(z >= 0, z, a_ref[...] * z)
    xw2 = jnp.dot(h, w_ref[...], preferred_element_type=jnp.float32)
    xw2_ref[...] = xw2
    y2_ref[...] = db * xw2


def _tc_mid(accp, db, xw1, b1, a_b, W2):
    return pl.pallas_call(
        _tc_mid_body,
        grid=(NBLK,),
        in_specs=[
            pl.BlockSpec((NC, ROWS_BLK, D), lambda i: (0, i, 0)),
            pl.BlockSpec((ROWS_BLK, D), lambda i: (i, 0)),
            pl.BlockSpec((ROWS_BLK, D), lambda i: (i, 0)),
            pl.BlockSpec((1, D), lambda i: (0, 0)),
            pl.BlockSpec((1, D), lambda i: (0, 0)),
            pl.BlockSpec((D, D), lambda i: (0, 0)),
        ],
        out_specs=[pl.BlockSpec((ROWS_BLK, D), lambda i: (i, 0))] * 2,
        out_shape=[jax.ShapeDtypeStruct((N_NODES, D), jnp.float32)] * 2,
    )(accp, db, xw1, b1, a_b, W2)


def _tc_out_body(accp_ref, db_ref, xw2_ref, b_ref, a_ref, out_ref):
    acc = accp_ref[0] + accp_ref[1]
    db = db_ref[...]
    z = db * acc + db * db * xw2_ref[...] + b_ref[...]
    out_ref[...] = jnp.where(z >= 0, z, a_ref[...] * z)


def _tc_out(accp, db, xw2, b2, a_b):
    return pl.pallas_call(
        _tc_out_body,
        grid=(NBLK,),
        in_specs=[
            pl.BlockSpec((NC, ROWS_BLK, D), lambda i: (0, i, 0)),
            pl.BlockSpec((ROWS_BLK, D), lambda i: (i, 0)),
            pl.BlockSpec((ROWS_BLK, D), lambda i: (i, 0)),
            pl.BlockSpec((1, D), lambda i: (0, 0)),
            pl.BlockSpec((1, D), lambda i: (0, 0)),
        ],
        out_specs=pl.BlockSpec((ROWS_BLK, D), lambda i: (i, 0)),
        out_shape=jax.ShapeDtypeStruct((N_NODES, D), jnp.float32),
    )(accp, db, xw2, b2, a_b)


def kernel(x, edge_index, W1, b1, W2, b2, prelu_a):
    src = edge_index[0]
    dst = edge_index[1]
    b1r = jnp.reshape(b1, (1, D))
    b2r = jnp.reshape(b2, (1, D))
    a_b = jnp.broadcast_to(jnp.reshape(prelu_a, (1, 1)), (1, D))

    degp = _sc_deg(dst)
    xw1, y1, db = _tc_a(x, W1, degp)
    accp1 = _sc_scatter(y1, src, dst)
    xw2, y2 = _tc_mid(accp1, db, xw1, b1r, a_b, W2)
    accp2 = _sc_scatter(y2, src, dst)
    return _tc_out(accp2, db, xw2, b2r, a_b)
